# baseline (speedup 1.0000x reference)
import functools

import jax
import jax.numpy as jnp
from jax import lax
from jax.experimental import pallas as pl
from jax.experimental.pallas import tpu as pltpu
from jax.experimental.pallas import tpu_sc as plsc

VOCAB = 100000
HIDDEN = 1024
BATCH = 4
SEQ = 8192
TOTAL = BATCH * SEQ

NUM_CORES = 2
PER_CORE = TOTAL // NUM_CORES
BLK = 1024
NBLK = PER_CORE // BLK
GROUP = 8
NGROUP = BLK // GROUP


_mesh = plsc.ScalarSubcoreMesh(axis_name="c")


@functools.partial(
    pl.kernel,
    out_type=jax.ShapeDtypeStruct((TOTAL, HIDDEN), jnp.float32),
    mesh=_mesh,
    scratch_types=[
        pltpu.SMEM((BLK,), jnp.int32),
        *[pltpu.SemaphoreType.DMA for _ in range(2 * GROUP)],
    ],
)
def _embed_gather_scs(table_hbm, idx_hbm, out_hbm, idx_s, *sems):
    cid = lax.axis_index("c")
    base = cid * PER_CORE

    @pl.loop(0, NBLK)
    def _block(blk):
        bbase = base + blk * BLK
        pltpu.sync_copy(idx_hbm.at[pl.ds(bbase, BLK)], idx_s)

        def row_desc(r, bank, k):
            row = idx_s[r]
            return pltpu.make_async_copy(
                table_hbm.at[pl.ds(row, 1)],
                out_hbm.at[pl.ds(bbase + r, 1)],
                sems[bank * GROUP + k],
            )

        def fire(g, bank):
            for k in range(GROUP):
                row_desc(g * GROUP + k, bank, k).start()

        def drain(g, bank):
            for k in range(GROUP):
                row_desc(g * GROUP + k, bank, k).wait()

        fire(0, 0)
        fire(1, 1)

        @pl.loop(0, NGROUP - 2, step=2)
        def _step(i):
            for j in range(2):
                g = i + j
                drain(g, j)
                fire(g + 2, j)

        drain(NGROUP - 2, 0)
        drain(NGROUP - 1, 1)


def kernel(input_ids, embedding):
    idx = input_ids.reshape(TOTAL).astype(jnp.int32)
    out = _embed_gather_scs(embedding, idx)
    return out.reshape(BATCH, SEQ, HIDDEN)


# speedup vs baseline: 36.3981x; 36.3981x over previous
import functools

import jax
import jax.numpy as jnp
from jax import lax
from jax.experimental import pallas as pl
from jax.experimental.pallas import tpu as pltpu
from jax.experimental.pallas import tpu_sc as plsc

VOCAB = 100000
HIDDEN = 1024
BATCH = 4
SEQ = 8192
TOTAL = BATCH * SEQ

NUM_CORES = 2
NUM_SUBCORES = 16
NUM_WORKERS = NUM_CORES * NUM_SUBCORES
PER_WORKER = TOTAL // NUM_WORKERS

CHUNK = 16
NCHUNK = PER_WORKER // CHUNK
NBUF = 4


_mesh = plsc.VectorSubcoreMesh(core_axis_name="c", subcore_axis_name="s")


@functools.partial(
    pl.kernel,
    out_type=jax.ShapeDtypeStruct((TOTAL, HIDDEN), jnp.float32),
    mesh=_mesh,
    scratch_types=[
        pltpu.VMEM((PER_WORKER,), jnp.int32),
        *[pltpu.VMEM((CHUNK, HIDDEN), jnp.float32) for _ in range(NBUF)],
        *[pltpu.SemaphoreType.DMA for _ in range(2 * NBUF)],
    ],
)
def _embed_gather(table_hbm, idx_hbm, out_hbm, idx_v, *bufs_and_sems):
    rows_v = bufs_and_sems[:NBUF]
    gsem = bufs_and_sems[NBUF : 2 * NBUF]
    ssem = bufs_and_sems[2 * NBUF :]
    wid = lax.axis_index("s") * NUM_CORES + lax.axis_index("c")
    base = wid * PER_WORKER
    pltpu.sync_copy(idx_hbm.at[pl.ds(base, PER_WORKER)], idx_v)

    def gather_desc(chunk, b):
        off = pl.multiple_of(chunk * CHUNK, 8)
        return pltpu.make_async_copy(
            table_hbm.at[idx_v.at[pl.ds(off, CHUNK)]], rows_v[b], gsem[b]
        )

    def scatter_desc(chunk, b):
        off = pl.multiple_of(chunk * CHUNK, 8)
        return pltpu.make_async_copy(
            rows_v[b], out_hbm.at[pl.ds(base + off, CHUNK)], ssem[b]
        )

    for b in range(NBUF):
        gather_desc(b, b).start()

    for r in range(2):
        gather_desc(r, r).wait()
        scatter_desc(r, r).start()

    @pl.loop(0, NCHUNK - NBUF, step=NBUF)
    def _round(i):
        for j in range(NBUF):
            r = i + j + 2
            b = (j + 2) % NBUF
            b2 = j
            scatter_desc(r - 2, b2).wait()
            gather_desc(r + NBUF - 2, b2).start()
            gather_desc(r, b).wait()
            scatter_desc(r, b).start()

    for r in range(NCHUNK - NBUF + 2, NCHUNK):
        b = r % NBUF
        gather_desc(r, b).wait()
        scatter_desc(r, b).start()

    for r in range(NCHUNK - NBUF, NCHUNK):
        scatter_desc(r, r % NBUF).wait()


def kernel(input_ids, embedding):
    idx = input_ids.reshape(TOTAL).astype(jnp.int32)
    out = _embed_gather(embedding, idx)
    return out.reshape(BATCH, SEQ, HIDDEN)
